# Initial kernel scaffold; baseline (speedup 1.0000x reference)
import jax
import jax.numpy as jnp
from jax.experimental import pallas as pl


def kernel(a_joint, c1, c2, tau):
    raise NotImplementedError("write your pallas kernel here")


# speedup vs baseline: 78.0274x; 78.0274x over previous
import functools

import jax
import jax.numpy as jnp
from jax import lax
from jax.experimental import pallas as pl
from jax.experimental.pallas import tpu as pltpu
from jax.experimental.pallas import tpu_sc as plsc

_B = 1024
_A = 100
_ACT = 1000
_NC, _NS = 2, 16
_NW = _NC * _NS
_RPW = _B // _NW
_AP = 112
_NG = _AP // 16
_CNT = 1024


def _sc_body(a_hbm, c1_hbm, c2_hbm, tau_hbm, out_hbm,
             a_v, o_v, cnt_v, c1_v, c2_v, tau_v):
    w = lax.axis_index("s") * _NC + lax.axis_index("c")
    base = w * (_RPW * _AP)
    pltpu.sync_copy(a_hbm.at[pl.ds(base, _RPW * _AP)], a_v)
    pltpu.sync_copy(c1_hbm, c1_v)
    pltpu.sync_copy(c2_hbm, c2_v)
    pltpu.sync_copy(tau_hbm, tau_v)

    zero16 = jnp.zeros((16,), jnp.float32)
    one16 = jnp.ones((16,), jnp.float32)
    for i in range(_CNT // 16):
        cnt_v[pl.ds(16 * i, 16)] = zero16

    def row(r, carry):
        ab = r * _AP
        idx = [a_v[pl.ds(ab + 16 * g, 16)] for g in range(_NG)]
        for g in range(_NG):
            plsc.addupdate_scatter(cnt_v, [idx[g]], one16)
        for g in range(_NG):
            ld = plsc.load_gather(cnt_v, [idx[g]])
            c1g = plsc.load_gather(c1_v, [idx[g]])
            c2g = plsc.load_gather(c2_v, [idx[g]])
            tg = plsc.load_gather(tau_v, [idx[g]])
            o_v[pl.ds(ab + 16 * g, 16)] = ld * (tg - c1g - c2g * ld)
        for g in range(_NG):
            plsc.store_scatter(cnt_v, [idx[g]], zero16)
        return carry

    lax.fori_loop(0, _RPW, row, 0)
    pltpu.sync_copy(o_v, out_hbm.at[pl.ds(base, _RPW * _AP)])


@jax.jit
def kernel(a_joint, c1, c2, tau):
    a32 = a_joint.astype(jnp.int32)
    pad = jnp.broadcast_to(
        jnp.arange(_ACT, _ACT + (_AP - _A), dtype=jnp.int32), (_B, _AP - _A))
    a_pad = jnp.concatenate([a32, pad], axis=1).reshape(-1)
    c1p = jnp.pad(c1, (0, _CNT - _ACT))
    c2p = jnp.pad(c2, (0, _CNT - _ACT))
    taup = jnp.pad(tau, (0, _CNT - _ACT))

    mesh = plsc.VectorSubcoreMesh(
        core_axis_name="c", subcore_axis_name="s",
        num_cores=_NC, num_subcores=_NS)
    out = pl.kernel(
        _sc_body,
        out_type=jax.ShapeDtypeStruct((_B * _AP,), jnp.float32),
        mesh=mesh,
        compiler_params=pltpu.CompilerParams(needs_layout_passes=False),
        scratch_types=[
            pltpu.VMEM((_RPW * _AP,), jnp.int32),
            pltpu.VMEM((_RPW * _AP,), jnp.float32),
            pltpu.VMEM((_CNT,), jnp.float32),
            pltpu.VMEM((_CNT,), jnp.float32),
            pltpu.VMEM((_CNT,), jnp.float32),
            pltpu.VMEM((_CNT,), jnp.float32),
        ],
    )(a_pad, c1p, c2p, taup)
    return out.reshape(_B, _AP)[:, :_A]
